# baseline (speedup 1.0000x reference)
import functools

import jax
import jax.numpy as jnp
from jax import lax
from jax.experimental import pallas as pl
from jax.experimental.pallas import tpu as pltpu
from jax.experimental.pallas import tpu_sc as plsc

D = 64
SEG = 4
L = 16
NC = 2
NS = 16
NW = NC * NS
CH = 64
MARGIN = 1.0


def _make_sc_kernel(B: int):
    per_w = B // NW
    n_chunks = per_w // CH

    mesh = plsc.VectorSubcoreMesh(core_axis_name="c", subcore_axis_name="s")

    row_buf = pltpu.VMEM((CH, D), jnp.float32)
    idx_buf = pltpu.VMEM((CH,), jnp.int32)

    @functools.partial(
        pl.kernel,
        out_type=jax.ShapeDtypeStruct((NW, L), jnp.float32),
        mesh=mesh,
        compiler_params=pltpu.CompilerParams(
            needs_layout_passes=False, use_tc_tiling_on_sc=False),
        scratch_types=[
            idx_buf, idx_buf, idx_buf,
            idx_buf, idx_buf, idx_buf,
            row_buf, row_buf, row_buf, row_buf, row_buf, row_buf,
            row_buf, row_buf, row_buf, row_buf, row_buf, row_buf,
            pltpu.VMEM((L * (L + 1),), jnp.float32),
            pltpu.VMEM((L * (L + 1),), jnp.float32),
            pltpu.VMEM((L,), jnp.float32),
            pltpu.SemaphoreType.DMA,
        ],
    )
    def sc_kernel(pos_hbm, neg_hbm, er_hbm, ei_hbm, rr_hbm, ri_hbm, out_hbm,
                  iph, ipr, ipt, inh, inr, int_,
                  p_hr, p_hi, p_rr, p_ri, p_tr, p_ti,
                  n_hr, n_hi, n_rr, n_ri, n_tr, n_ti,
                  spad_p, spad_n, lacc, gsem):
        wid = lax.axis_index("s") * NC + lax.axis_index("c")
        lacc[...] = jnp.zeros((L,), jnp.float32)

        iot = lax.iota(jnp.int32, L)

        def score_group(bufs, spad, g):
            hr_b, hi_b, rr_b, ri_b, tr_b, ti_b = bufs
            for e in range(L):
                row = g * L + e
                sv = None
                for s in range(SEG):
                    dd = pl.ds(L * s, L)
                    hr = hr_b[row, dd]
                    hi = hi_b[row, dd]
                    rr = rr_b[row, dd]
                    ri = ri_b[row, dd]
                    tr = tr_b[row, dd]
                    ti = ti_b[row, dd]
                    t = hr * (rr * tr + ri * ti) + hi * (rr * ti - ri * tr)
                    sv = t if sv is None else sv + t
                spad[pl.ds(e * (L + 1), L)] = sv
            acc = None
            for c in range(L):
                col = plsc.load_gather(spad, [iot * (L + 1) + c])
                acc = col if acc is None else acc + col
            return acc

        def chunk_body(c, carry):
            base = wid * per_w + c * CH
            sl = pl.ds(base, CH)
            pltpu.sync_copy(pos_hbm.at[0, sl], iph)
            pltpu.sync_copy(pos_hbm.at[1, sl], ipr)
            pltpu.sync_copy(pos_hbm.at[2, sl], ipt)
            pltpu.sync_copy(neg_hbm.at[0, sl], inh)
            pltpu.sync_copy(neg_hbm.at[1, sl], inr)
            pltpu.sync_copy(neg_hbm.at[2, sl], int_)
            cps = [
                pltpu.async_copy(er_hbm.at[iph], p_hr, gsem),
                pltpu.async_copy(ei_hbm.at[iph], p_hi, gsem),
                pltpu.async_copy(rr_hbm.at[ipr], p_rr, gsem),
                pltpu.async_copy(ri_hbm.at[ipr], p_ri, gsem),
                pltpu.async_copy(er_hbm.at[ipt], p_tr, gsem),
                pltpu.async_copy(ei_hbm.at[ipt], p_ti, gsem),
                pltpu.async_copy(er_hbm.at[inh], n_hr, gsem),
                pltpu.async_copy(ei_hbm.at[inh], n_hi, gsem),
                pltpu.async_copy(rr_hbm.at[inr], n_rr, gsem),
                pltpu.async_copy(ri_hbm.at[inr], n_ri, gsem),
                pltpu.async_copy(er_hbm.at[int_], n_tr, gsem),
                pltpu.async_copy(ei_hbm.at[int_], n_ti, gsem),
            ]
            for cp in cps:
                cp.wait()

            def group_body(g, carry2):
                ps = score_group((p_hr, p_hi, p_rr, p_ri, p_tr, p_ti),
                                 spad_p, g)
                ns = score_group((n_hr, n_hi, n_rr, n_ri, n_tr, n_ti),
                                 spad_n, g)
                dv = ns - ps + MARGIN
                lacc[...] = lacc[...] + jnp.maximum(dv, 0.0)
                return carry2

            return lax.fori_loop(0, CH // L, group_body, carry)

        lax.fori_loop(0, n_chunks, chunk_body, 0)
        pltpu.sync_copy(lacc, out_hbm.at[wid])

    return sc_kernel


def kernel(pos_exmpl, neg_exmpl, ent_real, ent_imag, rel_real, rel_imag):
    B = pos_exmpl.shape[1]
    sc = _make_sc_kernel(B)
    partials = sc(pos_exmpl, neg_exmpl, ent_real, ent_imag,
                  rel_real, rel_imag)
    return jnp.sum(partials)


# speedup vs baseline: 1.1600x; 1.1600x over previous
import functools

import jax
import jax.numpy as jnp
from jax import lax
from jax.experimental import pallas as pl
from jax.experimental.pallas import tpu as pltpu
from jax.experimental.pallas import tpu_sc as plsc

D = 64
SEG = 4
L = 16
NC = 2
NS = 16
NW = NC * NS
CH = 128
MARGIN = 1.0


def _make_sc_kernel(B: int):
    per_w = B // NW
    n_chunks = per_w // CH

    mesh = plsc.VectorSubcoreMesh(core_axis_name="c", subcore_axis_name="s")

    row_buf = pltpu.VMEM((CH, 2 * D), jnp.float32)
    idx_buf = pltpu.VMEM((CH,), jnp.int32)

    @functools.partial(
        pl.kernel,
        out_type=jax.ShapeDtypeStruct((NW, L), jnp.float32),
        mesh=mesh,
        compiler_params=pltpu.CompilerParams(
            needs_layout_passes=False, use_tc_tiling_on_sc=True),
        scratch_types=[
            idx_buf, idx_buf, idx_buf,
            idx_buf, idx_buf, idx_buf,
            row_buf, row_buf, row_buf,
            row_buf, row_buf, row_buf,
            pltpu.VMEM((L * (L + 1),), jnp.float32),
            pltpu.VMEM((L * (L + 1),), jnp.float32),
            pltpu.VMEM((L,), jnp.float32),
            pltpu.SemaphoreType.DMA,
        ],
    )
    def sc_kernel(ph_hbm, pr_hbm, pt_hbm, nh_hbm, nr_hbm, nt_hbm,
                  ent_hbm, rel_hbm, out_hbm,
                  iph, ipr, ipt, inh, inr, int_,
                  p_h, p_r, p_t, n_h, n_r, n_t,
                  spad_p, spad_n, lacc, gsem):
        wid = lax.axis_index("s") * NC + lax.axis_index("c")
        lacc[...] = jnp.zeros((L,), jnp.float32)

        iot = lax.iota(jnp.int32, L)

        def score_group(bufs, spad, g):
            h_b, r_b, t_b = bufs
            for e in range(L):
                row = g * L + e
                sv = None
                for s in range(SEG):
                    dr = pl.ds(L * s, L)
                    di = pl.ds(D + L * s, L)
                    hr = h_b[row, dr]
                    hi = h_b[row, di]
                    rr = r_b[row, dr]
                    ri = r_b[row, di]
                    tr = t_b[row, dr]
                    ti = t_b[row, di]
                    t = hr * (rr * tr + ri * ti) + hi * (rr * ti - ri * tr)
                    sv = t if sv is None else sv + t
                spad[pl.ds(e * (L + 1), L)] = sv
            acc = None
            for c in range(L):
                col = plsc.load_gather(spad, [iot * (L + 1) + c])
                acc = col if acc is None else acc + col
            return acc

        def chunk_body(c, carry):
            base = wid * per_w + c * CH
            sl = pl.ds(base, CH)
            pltpu.sync_copy(ph_hbm.at[sl], iph)
            pltpu.sync_copy(pr_hbm.at[sl], ipr)
            pltpu.sync_copy(pt_hbm.at[sl], ipt)
            pltpu.sync_copy(nh_hbm.at[sl], inh)
            pltpu.sync_copy(nr_hbm.at[sl], inr)
            pltpu.sync_copy(nt_hbm.at[sl], int_)
            cps = [
                pltpu.async_copy(ent_hbm.at[iph], p_h, gsem),
                pltpu.async_copy(rel_hbm.at[ipr], p_r, gsem),
                pltpu.async_copy(ent_hbm.at[ipt], p_t, gsem),
                pltpu.async_copy(ent_hbm.at[inh], n_h, gsem),
                pltpu.async_copy(rel_hbm.at[inr], n_r, gsem),
                pltpu.async_copy(ent_hbm.at[int_], n_t, gsem),
            ]
            for cp in cps:
                cp.wait()

            def group_body(g, carry2):
                ps = score_group((p_h, p_r, p_t), spad_p, g)
                ns = score_group((n_h, n_r, n_t), spad_n, g)
                dv = ns - ps + MARGIN
                lacc[...] = lacc[...] + jnp.maximum(dv, 0.0)
                return carry2

            return lax.fori_loop(0, CH // L, group_body, carry)

        lax.fori_loop(0, n_chunks, chunk_body, 0)
        pltpu.sync_copy(lacc, out_hbm.at[wid])

    return sc_kernel


def kernel(pos_exmpl, neg_exmpl, ent_real, ent_imag, rel_real, rel_imag):
    B = pos_exmpl.shape[1]
    ent_ri = jnp.concatenate([ent_real, ent_imag], axis=1)
    rel_ri = jnp.concatenate([rel_real, rel_imag], axis=1)
    sc = _make_sc_kernel(B)
    partials = sc(pos_exmpl[0], pos_exmpl[1], pos_exmpl[2],
                  neg_exmpl[0], neg_exmpl[1], neg_exmpl[2],
                  ent_ri, rel_ri)
    return jnp.sum(partials)
